# Initial kernel scaffold; baseline (speedup 1.0000x reference)
import functools
import math

import jax
import jax.numpy as jnp
from jax import lax
from jax.experimental import pallas as pl
from jax.experimental.pallas import tpu as pltpu
from jax.experimental.pallas import tpu_sc as plsc

EMB_D = 32
SCALE = math.sqrt(32.0)

NUM_CORES = 2
NUM_SUBCORES = 16
NW = NUM_CORES * NUM_SUBCORES

TOKENS_B = 4096
TOKENS_T = 200
B_TOTAL = TOKENS_B * TOKENS_T
PER_W = B_TOTAL // NW
SLAB = 128
NSLAB = PER_W // SLAB


def _build():
    mesh = plsc.VectorSubcoreMesh(core_axis_name="c", subcore_axis_name="s")

    @functools.partial(
        pl.kernel,
        mesh=mesh,
        out_type=jax.ShapeDtypeStruct((NW, NSLAB, SLAB, EMB_D), jnp.float32),
        scratch_types=[
            pltpu.VMEM((NSLAB, SLAB), jnp.int32),
            pltpu.VMEM((SLAB, EMB_D), jnp.float32),
            pltpu.SemaphoreType.DMA,
        ],
    )
    def emb_kernel(idx_hbm, table_hbm, out_hbm, idx_v, rows_v, gsem):
        wid = lax.axis_index("s") * NUM_CORES + lax.axis_index("c")
        pltpu.sync_copy(idx_hbm.at[wid], idx_v)

        def slab_body(s, carry):
            pltpu.async_copy(table_hbm.at[idx_v.at[s]], rows_v, gsem).wait()

            def row_body(i, c2):
                for col in (0, 16):
                    sl = pl.ds(col, 16)
                    rows_v[i, sl] = rows_v[i, sl] * SCALE
                return c2

            lax.fori_loop(0, SLAB, row_body, 0, unroll=4)
            pltpu.sync_copy(rows_v, out_hbm.at[wid, s])
            return carry

        lax.fori_loop(0, NSLAB, slab_body, 0)

    return emb_kernel


_emb_kernel = _build()


def kernel(tokens, embedding_weight):
    idx = tokens.astype(jnp.int32).reshape(NW, NSLAB, SLAB)
    out = _emb_kernel(idx, embedding_weight)
    return out.reshape(TOKENS_B, TOKENS_T, EMB_D)


# speedup vs baseline: 1.2557x; 1.2557x over previous
import functools
import math

import jax
import jax.numpy as jnp
from jax import lax
from jax.experimental import pallas as pl
from jax.experimental.pallas import tpu as pltpu
from jax.experimental.pallas import tpu_sc as plsc

EMB_D = 32
SCALE = math.sqrt(32.0)

NUM_CORES = 2
NUM_SUBCORES = 16
NW = NUM_CORES * NUM_SUBCORES

TOKENS_B = 4096
TOKENS_T = 200
B_TOTAL = TOKENS_B * TOKENS_T
PER_W = B_TOTAL // NW
SLAB = 128
NSLAB = PER_W // SLAB


def _build():
    mesh = plsc.VectorSubcoreMesh(core_axis_name="c", subcore_axis_name="s")

    @functools.partial(
        pl.kernel,
        mesh=mesh,
        compiler_params=pltpu.CompilerParams(use_tc_tiling_on_sc=False),
        out_type=jax.ShapeDtypeStruct((NW, NSLAB, SLAB, EMB_D), jnp.float32),
        scratch_types=[
            pltpu.VMEM((NSLAB, SLAB), jnp.int32),
            pltpu.VMEM((SLAB, EMB_D), jnp.float32),
            pltpu.SemaphoreType.DMA,
        ],
    )
    def emb_kernel(idx_hbm, table_hbm, out_hbm, idx_v, rows_v, gsem):
        wid = lax.axis_index("s") * NUM_CORES + lax.axis_index("c")
        pltpu.sync_copy(idx_hbm.at[wid], idx_v)

        def slab_body(s, carry):
            pltpu.async_copy(table_hbm.at[idx_v.at[s]], rows_v, gsem).wait()

            def row_body(i, c2):
                for col in (0, 16):
                    sl = pl.ds(col, 16)
                    rows_v[i, sl] = rows_v[i, sl] * SCALE
                return c2

            lax.fori_loop(0, SLAB, row_body, 0, unroll=4)
            pltpu.sync_copy(rows_v, out_hbm.at[wid, s])
            return carry

        lax.fori_loop(0, NSLAB, slab_body, 0)

    return emb_kernel


_emb_kernel = _build()


def kernel(tokens, embedding_weight):
    idx = tokens.astype(jnp.int32).reshape(NW, NSLAB, SLAB)
    out = _emb_kernel(idx, embedding_weight)
    return out.reshape(TOKENS_B, TOKENS_T, EMB_D)


# speedup vs baseline: 1.4794x; 1.1782x over previous
import functools
import math

import jax
import jax.numpy as jnp
from jax import lax
from jax.experimental import pallas as pl
from jax.experimental.pallas import tpu as pltpu
from jax.experimental.pallas import tpu_sc as plsc

EMB_D = 32
SCALE = math.sqrt(32.0)

NUM_CORES = 2
NUM_SUBCORES = 16
NW = NUM_CORES * NUM_SUBCORES

TOKENS_B = 4096
TOKENS_T = 200
B_TOTAL = TOKENS_B * TOKENS_T
PER_W = B_TOTAL // NW
SLAB = 128
NSLAB = PER_W // SLAB

RB = 8
LOOKAHEAD = 4
NGROUP = NSLAB // RB


def _build():
    mesh = plsc.VectorSubcoreMesh(core_axis_name="c", subcore_axis_name="s")

    @functools.partial(
        pl.kernel,
        mesh=mesh,
        compiler_params=pltpu.CompilerParams(use_tc_tiling_on_sc=False),
        out_type=jax.ShapeDtypeStruct((NW, NSLAB, SLAB, EMB_D), jnp.float32),
        scratch_types=[
            pltpu.VMEM((NSLAB, SLAB), jnp.int32),
            pltpu.VMEM((RB, SLAB, EMB_D), jnp.float32),
            pltpu.SemaphoreType.DMA,
            pltpu.SemaphoreType.DMA,
        ],
    )
    def emb_kernel(idx_hbm, table_hbm, out_hbm, idx_v, rows_v, gsem, ssem):
        wid = lax.axis_index("s") * NUM_CORES + lax.axis_index("c")
        pltpu.sync_copy(idx_hbm.at[wid], idx_v)

        def fire_gather(slab, buf):
            pltpu.async_copy(table_hbm.at[idx_v.at[slab]], rows_v.at[buf], gsem)

        def wait_gather(slab, buf):
            pltpu.make_async_copy(
                table_hbm.at[idx_v.at[slab]], rows_v.at[buf], gsem
            ).wait()

        def fire_scatter(slab, buf):
            pltpu.async_copy(rows_v.at[buf], out_hbm.at[wid, slab], ssem)

        def wait_scatter(slab, buf):
            pltpu.make_async_copy(
                rows_v.at[buf], out_hbm.at[wid, slab], ssem
            ).wait()

        for z in range(LOOKAHEAD):
            fire_gather(z, z)

        def group_body(t, carry):
            for b in range(RB):
                s = t * RB + b
                buf_g = (b + LOOKAHEAD) % RB

                if b >= LOOKAHEAD:
                    wait_scatter(s - LOOKAHEAD, buf_g)
                else:
                    @pl.when(t > 0)
                    def _():
                        wait_scatter(s - LOOKAHEAD, buf_g)

                if b < RB - LOOKAHEAD:
                    fire_gather(s + LOOKAHEAD, buf_g)
                else:
                    @pl.when(t < NGROUP - 1)
                    def _():
                        fire_gather(s + LOOKAHEAD, buf_g)

                wait_gather(s, b)
                rv = rows_v.at[b]

                def scale_body(i, c2):
                    base = i * 8
                    for r in range(8):
                        for col in (0, 16):
                            sl = pl.ds(col, 16)
                            rv[base + r, sl] = rv[base + r, sl] * SCALE
                    return c2

                lax.fori_loop(0, SLAB // 8, scale_body, 0)
                fire_scatter(s, b)
            return carry

        lax.fori_loop(0, NGROUP, group_body, 0)

        for z in range(LOOKAHEAD):
            s = NSLAB - LOOKAHEAD + z
            wait_scatter(s, s % RB)

    return emb_kernel


_emb_kernel = _build()


def kernel(tokens, embedding_weight):
    idx = tokens.astype(jnp.int32).reshape(NW, NSLAB, SLAB)
    out = _emb_kernel(idx, embedding_weight)
    return out.reshape(TOKENS_B, TOKENS_T, EMB_D)
